# Initial kernel scaffold; baseline (speedup 1.0000x reference)
import jax
import jax.numpy as jnp
from jax import lax
from jax.experimental import pallas as pl
from jax.experimental.pallas import tpu as pltpu
from jax.experimental.pallas import tpu_sc as plsc

N = 10000
E = 160000
D_IN = 256
D_HID = 256
D_OUT = 128

NC = 2
NT = 16
LW = 128
ROWS = 1280
EPAD = ROWS * LW
RPT = ROWS // NT
NP = 10016
SLC = NP // NT
BN = 512

_mesh = plsc.VectorSubcoreMesh(core_axis_name="core", subcore_axis_name="subcore")


def _node_chunks(total):
    out = []
    left = total
    while left > 0:
        c = min(128, left)
        out.append(c)
        left -= c
    return out


def _k1_body(x_ref, w_ref, b_ref, o_ref):
    h = lax.dot_general(x_ref[...], w_ref[...], (((1,), (1,)), ((), ())),
                        preferred_element_type=jnp.float32)
    h = h + b_ref[...][None, :]
    for k in range(4):
        o_ref[k] = h[:, k * 128:(k + 1) * 128]


def _k1(x, wcat, bcat):
    grid = (NP + BN - 1) // BN
    return pl.pallas_call(
        _k1_body,
        grid=(grid,),
        in_specs=[
            pl.BlockSpec((BN, D_IN), lambda i: (i, 0)),
            pl.BlockSpec((512, D_IN), lambda i: (0, 0)),
            pl.BlockSpec((512,), lambda i: (0,)),
        ],
        out_specs=pl.BlockSpec((4, BN, 128), lambda i: (0, i, 0)),
        out_shape=jax.ShapeDtypeStruct((4, NP, 128), jnp.float32),
    )(x, wcat, bcat)


def _sc1_kernel(ei_ref, h1_ref, deg_ref, agg_ref,
                sidx, didx, gbuf, z16, o16, acc, acc16):
    cid = lax.axis_index("core")
    sid = lax.axis_index("subcore")
    r0 = sid * SLC
    er0 = sid * RPT

    @pl.loop(0, 128)
    def _fill(r):
        z16[r] = jnp.zeros((16,), jnp.float32)
        o16[r] = jnp.ones((16,), jnp.float32)

    off = 0
    for c in _node_chunks(SLC):
        pltpu.sync_copy(z16.at[pl.ds(0, c)], acc16.at[pl.ds(r0 + off, c)])
        off += c
    plsc.subcore_barrier()
    pltpu.sync_copy(ei_ref.at[1 - cid].at[pl.ds(er0, RPT)], didx)

    @pl.loop(0, RPT)
    def _deg(g):
        pltpu.sync_copy(o16, acc16.at[didx.at[g]], add=True)

    plsc.subcore_barrier()
    pltpu.sync_copy(acc16.at[pl.ds(r0, SLC)], deg_ref.at[cid].at[pl.ds(r0, SLC)])

    for p, (sp, dp) in enumerate(((0, 1), (1, 0))):
        tk = 2 * p + cid
        pltpu.sync_copy(h1_ref.at[tk].at[pl.ds(r0, SLC)], acc.at[pl.ds(r0, SLC)])
        plsc.subcore_barrier()
        pltpu.sync_copy(ei_ref.at[sp].at[pl.ds(er0, RPT)], sidx)
        pltpu.sync_copy(ei_ref.at[dp].at[pl.ds(er0, RPT)], didx)

        @pl.loop(0, RPT)
        def _agg(g):
            pltpu.sync_copy(h1_ref.at[tk].at[sidx.at[g]], gbuf)
            pltpu.sync_copy(gbuf, acc.at[didx.at[g]], add=True)

        plsc.subcore_barrier()
        pltpu.sync_copy(acc.at[pl.ds(r0, SLC)], agg_ref.at[tk].at[pl.ds(r0, SLC)])
        plsc.subcore_barrier()


def _sc1(ei_r, h1):
    return pl.kernel(
        _sc1_kernel,
        out_type=[
            jax.ShapeDtypeStruct((2, NP, 16), jnp.float32),
            jax.ShapeDtypeStruct((4, NP, 128), jnp.float32),
        ],
        mesh=_mesh,
        scratch_types=[
            pltpu.VMEM((RPT, LW), jnp.int32),
            pltpu.VMEM((RPT, LW), jnp.int32),
            pltpu.VMEM((LW, 128), jnp.float32),
            pltpu.VMEM((128, 16), jnp.float32),
            pltpu.VMEM((128, 16), jnp.float32),
            pltpu.VMEM_SHARED((NP, 128), jnp.float32),
            pltpu.VMEM_SHARED((NP, 16), jnp.float32),
        ],
    )(ei_r, h1)


def _k2_body(agg_ref, deg_ref, w_ref, b_ref, o_ref):
    a = jnp.concatenate([agg_ref[0], agg_ref[1]], axis=1)
    scale = 1.0 / (1.0 + deg_ref[0][:, 0:1])
    act = jnp.maximum(a * scale, 0.0)
    y = lax.dot_general(act, w_ref[0], (((1,), (1,)), ((), ())),
                        preferred_element_type=jnp.float32)
    y = y + b_ref[0][None, :]
    o_ref[0, 0] = y[:, :64]
    o_ref[0, 1] = y[:, 64:]


def _k2(agg1, deg, w2cat, b2cat):
    grid_i = (NP + BN - 1) // BN
    return pl.pallas_call(
        _k2_body,
        grid=(2, grid_i),
        in_specs=[
            pl.BlockSpec((2, BN, 128), lambda p, i: (p, i, 0)),
            pl.BlockSpec((1, BN, 16), lambda p, i: (p, i, 0)),
            pl.BlockSpec((1, D_OUT, D_HID), lambda p, i: (p, 0, 0)),
            pl.BlockSpec((1, D_OUT), lambda p, i: (p, 0)),
        ],
        out_specs=pl.BlockSpec((1, 2, BN, 64), lambda p, i: (p, 0, i, 0)),
        out_shape=jax.ShapeDtypeStruct((2, 2, NP, 64), jnp.float32),
    )(agg1, deg, w2cat, b2cat)


def _sc2_kernel(ei_ref, h2_ref, agg_ref, sidx, didx, gbuf, acc):
    cid = lax.axis_index("core")
    sid = lax.axis_index("subcore")
    r0 = sid * SLC
    er0 = sid * RPT

    for p, (sp, dp) in enumerate(((1, 0), (0, 1))):
        pltpu.sync_copy(h2_ref.at[p].at[cid].at[pl.ds(r0, SLC)],
                        acc.at[pl.ds(r0, SLC)])
        plsc.subcore_barrier()
        pltpu.sync_copy(ei_ref.at[sp].at[pl.ds(er0, RPT)], sidx)
        pltpu.sync_copy(ei_ref.at[dp].at[pl.ds(er0, RPT)], didx)

        @pl.loop(0, RPT)
        def _agg(g):
            pltpu.sync_copy(h2_ref.at[p].at[cid].at[sidx.at[g]], gbuf)
            pltpu.sync_copy(gbuf, acc.at[didx.at[g]], add=True)

        plsc.subcore_barrier()
        pltpu.sync_copy(acc.at[pl.ds(r0, SLC)],
                        agg_ref.at[p].at[cid].at[pl.ds(r0, SLC)])
        plsc.subcore_barrier()


def _sc2(ei_r, h2):
    return pl.kernel(
        _sc2_kernel,
        out_type=jax.ShapeDtypeStruct((2, 2, NP, 64), jnp.float32),
        mesh=_mesh,
        scratch_types=[
            pltpu.VMEM((RPT, LW), jnp.int32),
            pltpu.VMEM((RPT, LW), jnp.int32),
            pltpu.VMEM((LW, 64), jnp.float32),
            pltpu.VMEM_SHARED((NP, 64), jnp.float32),
        ],
    )(ei_r, h2)


def _k3_body(s_ref, t_ref, ds_ref, dt_ref, o_ref):
    s = jnp.concatenate([s_ref[0, 0], s_ref[0, 1]], axis=1)
    t = jnp.concatenate([t_ref[0, 0], t_ref[0, 1]], axis=1)
    s = s * (1.0 / (1.0 + ds_ref[0][:, 0:1]))
    t = t * (1.0 / (1.0 + dt_ref[0][:, 0:1]))
    m = lax.dot_general(s, t, (((1,), (1,)), ((), ())),
                        preferred_element_type=jnp.float32)
    o_ref[...] = 1.0 / (1.0 + jnp.exp(-m))


def _k3(agg2, deg):
    grid = (N + BN - 1) // BN
    return pl.pallas_call(
        _k3_body,
        grid=(grid, grid),
        in_specs=[
            pl.BlockSpec((1, 2, BN, 64), lambda i, j: (0, 0, i, 0)),
            pl.BlockSpec((1, 2, BN, 64), lambda i, j: (1, 0, j, 0)),
            pl.BlockSpec((1, BN, 16), lambda i, j: (1, i, 0)),
            pl.BlockSpec((1, BN, 16), lambda i, j: (0, j, 0)),
        ],
        out_specs=pl.BlockSpec((BN, BN), lambda i, j: (i, j)),
        out_shape=jax.ShapeDtypeStruct((N, N), jnp.float32),
    )(agg2, agg2, deg, deg)


def kernel(x, edge_index, edge_weight, W1s, b1s, W2s, b2s, W1t, b1t, W2t, b2t):
    del edge_weight
    pad = jnp.full((2, EPAD - E), N, dtype=edge_index.dtype)
    ei_r = jnp.concatenate([edge_index, pad], axis=1).reshape(2, ROWS, LW)

    wcat = jnp.concatenate([W1s, W1t], axis=0)
    bcat = jnp.concatenate([b1s, b1t], axis=0)
    w2cat = jnp.stack([W2s, W2t], axis=0)
    b2cat = jnp.stack([b2s, b2t], axis=0)

    h1 = _k1(x, wcat, bcat)
    deg, agg1 = _sc1(ei_r, h1)
    h2 = _k2(agg1, deg, w2cat, b2cat)
    agg2 = _sc2(ei_r, h2)
    return _k3(agg2, deg)


# speedup vs baseline: 5.0142x; 5.0142x over previous
import jax
import jax.numpy as jnp
from jax import lax
from jax.experimental import pallas as pl
from jax.experimental.pallas import tpu as pltpu
from jax.experimental.pallas import tpu_sc as plsc

N = 10000
E = 160000
D_IN = 256
D_HID = 256
D_OUT = 128

NC = 2
NT = 16
LW = 128
ROWS = 1280
EPAD = ROWS * LW
RPT = ROWS // NT
NP = 10240
SLC = NP // NT
BN = 512

_mesh = plsc.VectorSubcoreMesh(core_axis_name="core", subcore_axis_name="subcore")


def _k1_body(x_ref, w_ref, b_ref, o_ref):
    h = lax.dot_general(x_ref[...], w_ref[...], (((1,), (1,)), ((), ())),
                        preferred_element_type=jnp.float32)
    h = h + b_ref[...][None, :]
    for k in range(4):
        o_ref[k] = h[:, k * 128:(k + 1) * 128]


def _k1(x, wcat, bcat):
    grid = (NP + BN - 1) // BN
    return pl.pallas_call(
        _k1_body,
        grid=(grid,),
        in_specs=[
            pl.BlockSpec((BN, D_IN), lambda i: (i, 0)),
            pl.BlockSpec((512, D_IN), lambda i: (0, 0)),
            pl.BlockSpec((512,), lambda i: (0,)),
        ],
        out_specs=pl.BlockSpec((4, BN, 128), lambda i: (0, i, 0)),
        out_shape=jax.ShapeDtypeStruct((4, NP, 128), jnp.float32),
    )(x, wcat, bcat)


def _fill_const(buf, val):
    @pl.loop(0, LW)
    def _f(r):
        for q in range(8):
            buf[r, pl.ds(q * 16, 16)] = jnp.full((16,), val, jnp.float32)


def _scdeg_kernel(ei_ref, deg_ref, didx, gbuf, acc):
    cid = lax.axis_index("core")
    sid = lax.axis_index("subcore")
    r0 = sid * SLC
    er0 = sid * RPT

    _fill_const(gbuf, 0.0)
    for q in range(SLC // LW):
        pltpu.sync_copy(gbuf, acc.at[pl.ds(r0 + q * LW, LW)])
    _fill_const(gbuf, 1.0)
    plsc.subcore_barrier()
    pltpu.sync_copy(ei_ref.at[1 - cid].at[pl.ds(er0, RPT)], didx)

    @pl.loop(0, RPT)
    def _deg(g):
        pltpu.sync_copy(gbuf, acc.at[didx.at[g]], add=True)

    plsc.subcore_barrier()
    pltpu.sync_copy(acc.at[pl.ds(r0, SLC)], deg_ref.at[cid].at[pl.ds(r0, SLC)])


def _scdeg(ei_r):
    return pl.kernel(
        _scdeg_kernel,
        out_type=jax.ShapeDtypeStruct((2, NP, LW), jnp.float32),
        mesh=_mesh,
        scratch_types=[
            pltpu.VMEM((RPT, LW), jnp.int32),
            pltpu.VMEM((LW, LW), jnp.float32),
            pltpu.VMEM_SHARED((NP, LW), jnp.float32),
        ],
    )(ei_r)


def _sc1_kernel(ei_ref, h1_ref, agg_ref, sidx, didx, gbuf, acc):
    cid = lax.axis_index("core")
    sid = lax.axis_index("subcore")
    r0 = sid * SLC
    er0 = sid * RPT

    for p, (sp, dp) in enumerate(((0, 1), (1, 0))):
        tk = 2 * p + cid
        pltpu.sync_copy(h1_ref.at[tk].at[pl.ds(r0, SLC)], acc.at[pl.ds(r0, SLC)])
        plsc.subcore_barrier()
        pltpu.sync_copy(ei_ref.at[sp].at[pl.ds(er0, RPT)], sidx)
        pltpu.sync_copy(ei_ref.at[dp].at[pl.ds(er0, RPT)], didx)

        @pl.loop(0, RPT)
        def _agg(g):
            pltpu.sync_copy(h1_ref.at[tk].at[sidx.at[g]], gbuf)
            pltpu.sync_copy(gbuf, acc.at[didx.at[g]], add=True)

        plsc.subcore_barrier()
        pltpu.sync_copy(acc.at[pl.ds(r0, SLC)], agg_ref.at[tk].at[pl.ds(r0, SLC)])
        plsc.subcore_barrier()


def _sc1(ei_r, h1):
    return pl.kernel(
        _sc1_kernel,
        out_type=jax.ShapeDtypeStruct((4, NP, 128), jnp.float32),
        mesh=_mesh,
        scratch_types=[
            pltpu.VMEM((RPT, LW), jnp.int32),
            pltpu.VMEM((RPT, LW), jnp.int32),
            pltpu.VMEM((LW, 128), jnp.float32),
            pltpu.VMEM_SHARED((NP, 128), jnp.float32),
        ],
    )(ei_r, h1)


def _k2_body(agg_ref, deg_ref, w_ref, b_ref, o_ref):
    a = jnp.concatenate([agg_ref[0], agg_ref[1]], axis=1)
    scale = 1.0 / (1.0 + deg_ref[0][:, 0:1])
    act = jnp.maximum(a * scale, 0.0)
    y = lax.dot_general(act, w_ref[0], (((1,), (1,)), ((), ())),
                        preferred_element_type=jnp.float32)
    y = y + b_ref[0]
    o_ref[0] = y


def _k2(agg1, deg, w2cat, b2cat):
    grid_i = (NP + BN - 1) // BN
    return pl.pallas_call(
        _k2_body,
        grid=(2, grid_i),
        in_specs=[
            pl.BlockSpec((2, BN, 128), lambda p, i: (p, i, 0)),
            pl.BlockSpec((1, BN, LW), lambda p, i: (p, i, 0)),
            pl.BlockSpec((1, D_OUT, D_HID), lambda p, i: (p, 0, 0)),
            pl.BlockSpec((1, 1, D_OUT), lambda p, i: (p, 0, 0)),
        ],
        out_specs=pl.BlockSpec((1, BN, D_OUT), lambda p, i: (p, i, 0)),
        out_shape=jax.ShapeDtypeStruct((2, NP, D_OUT), jnp.float32),
    )(agg1, deg, w2cat, b2cat)


RPT2 = ROWS // (NC * NT)


def _sc2_kernel(ei_ref, h2_ref, agg_ref, sidx, didx, gbuf, acc):
    cid = lax.axis_index("core")
    sid = lax.axis_index("subcore")
    r0 = sid * SLC
    er0 = cid * (ROWS // NC) + sid * RPT2

    for p, (sp, dp) in enumerate(((1, 0), (0, 1))):
        _fill_const(gbuf, 0.0)

        @pl.when(cid == 0)
        def _():
            pltpu.sync_copy(h2_ref.at[p].at[pl.ds(r0, SLC)],
                            acc.at[pl.ds(r0, SLC)])

        @pl.when(cid != 0)
        def _():
            for q in range(SLC // LW):
                pltpu.sync_copy(gbuf, acc.at[pl.ds(r0 + q * LW, LW)])

        plsc.subcore_barrier()
        pltpu.sync_copy(ei_ref.at[sp].at[pl.ds(er0, RPT2)], sidx)
        pltpu.sync_copy(ei_ref.at[dp].at[pl.ds(er0, RPT2)], didx)

        @pl.loop(0, RPT2)
        def _agg(g):
            pltpu.sync_copy(h2_ref.at[p].at[sidx.at[g]], gbuf)
            pltpu.sync_copy(gbuf, acc.at[didx.at[g]], add=True)

        plsc.subcore_barrier()
        pltpu.sync_copy(acc.at[pl.ds(r0, SLC)],
                        agg_ref.at[p].at[cid].at[pl.ds(r0, SLC)])
        plsc.subcore_barrier()


def _sc2(ei_r, h2):
    return pl.kernel(
        _sc2_kernel,
        out_type=jax.ShapeDtypeStruct((2, 2, NP, D_OUT), jnp.float32),
        mesh=_mesh,
        scratch_types=[
            pltpu.VMEM((RPT2, LW), jnp.int32),
            pltpu.VMEM((RPT2, LW), jnp.int32),
            pltpu.VMEM((LW, D_OUT), jnp.float32),
            pltpu.VMEM_SHARED((NP, D_OUT), jnp.float32),
        ],
    )(ei_r, h2)


def _k3_body(s_ref, t_ref, ds_ref, dt_ref, o_ref):
    s = s_ref[0, 0] + s_ref[0, 1]
    t = t_ref[0, 0] + t_ref[0, 1]
    s = s * (1.0 / (1.0 + ds_ref[0][:, 0:1]))
    t = t * (1.0 / (1.0 + dt_ref[0][:, 0:1]))
    m = lax.dot_general(s, t, (((1,), (1,)), ((), ())),
                        preferred_element_type=jnp.float32)
    o_ref[...] = 1.0 / (1.0 + jnp.exp(-m))


def _k3(agg2, deg):
    grid = (N + BN - 1) // BN
    return pl.pallas_call(
        _k3_body,
        grid=(grid, grid),
        in_specs=[
            pl.BlockSpec((1, 2, BN, D_OUT), lambda i, j: (0, 0, i, 0)),
            pl.BlockSpec((1, 2, BN, D_OUT), lambda i, j: (1, 0, j, 0)),
            pl.BlockSpec((1, BN, LW), lambda i, j: (1, i, 0)),
            pl.BlockSpec((1, BN, LW), lambda i, j: (0, j, 0)),
        ],
        out_specs=pl.BlockSpec((BN, BN), lambda i, j: (i, j)),
        out_shape=jax.ShapeDtypeStruct((N, N), jnp.float32),
    )(agg2, agg2, deg, deg)


def kernel(x, edge_index, edge_weight, W1s, b1s, W2s, b2s, W1t, b1t, W2t, b2t):
    del edge_weight
    pad = jnp.full((2, EPAD - E), N, dtype=edge_index.dtype)
    ei_r = jnp.concatenate([edge_index, pad], axis=1).reshape(2, ROWS, LW)

    wcat = jnp.concatenate([W1s, W1t], axis=0)
    bcat = jnp.concatenate([b1s, b1t], axis=0)
    w2cat = jnp.stack([W2s, W2t], axis=0)
    b2cat = jnp.stack([b2s, b2t], axis=0).reshape(2, 1, D_OUT)

    h1 = _k1(x, wcat, bcat)
    deg = _scdeg(ei_r)
    agg1 = _sc1(ei_r, h1)
    h2 = _k2(agg1, deg, w2cat, b2cat)
    agg2 = _sc2(ei_r, h2)
    return _k3(agg2, deg)
